# baseline (speedup 1.0000x reference)
import functools

import jax
import jax.numpy as jnp
from jax.experimental import pallas as pl
from jax.experimental.pallas import tpu as pltpu

MARGIN = 1.0
BIG = 131072.0
NUM_LABELS = 64


def _dot_block(a_ref, c_ref, h_ref, idx, bc):
    a_blk = a_ref[pl.ds(idx * bc, bc), :]
    h_ref[...] = jax.lax.dot_general(
        c_ref[...], a_blk, (((1,), (1,)), ((), ())),
        preferred_element_type=jnp.float32)


def _reduce_block(h_ref, sqr_ref, cntr_ref, loss_ref, cnt_ref, idx, gate, bc, b):
    ht = h_ref[...]
    mx = jnp.max(ht, axis=0, keepdims=True)
    mn = jnp.min(ht, axis=0, keepdims=True)
    sq_blk = sqr_ref[:, pl.ds(idx * bc, bc)]
    cnt_blk = cntr_ref[:, pl.ds(idx * bc, bc)]
    pos_max = mx - BIG + sq_blk
    neg_min = mn + sq_blk
    valid = (cnt_blk >= 2.0) & (cnt_blk < float(b))
    vf = valid.astype(jnp.float32)
    losses = jnp.maximum(pos_max - neg_min + MARGIN, 0.0) * vf
    loss_ref[...] += jnp.where(gate, jnp.sum(losses), 0.0).reshape(1, 1)
    cnt_ref[...] += jnp.where(gate, jnp.sum(valid.astype(jnp.int32)), 0).reshape(1, 1)


def _triplet_kernel(emb_ref, trow_ref, loss_ref, cnt_ref,
                    a_ref, c_ref, h0_ref, h1_ref, sqr_ref, cntr_ref,
                    *, bc, nblk):
    i = pl.program_id(0)
    b = emb_ref.shape[0]

    @pl.when(i == 0)
    def _build():
        e = emb_ref[...]
        esq = e * e
        sq = jnp.sum(esq, axis=1, keepdims=True)
        tf = trow_ref[...].astype(jnp.float32)
        t = jnp.transpose(tf)
        lbl = jax.lax.broadcasted_iota(
            jnp.int32, (1, NUM_LABELS), 1).astype(jnp.float32)
        oh = (t == lbl).astype(jnp.float32)
        ones_col = jnp.ones((b, 1), jnp.float32)
        a_ref[...] = jnp.concatenate(
            [-2.0 * e, ones_col, BIG * oh], axis=1).astype(jnp.bfloat16)
        c_ref[...] = jnp.concatenate(
            [e, sq, oh], axis=1).astype(jnp.bfloat16)
        sqr_ref[...] = jax.lax.dot_general(
            jnp.ones((1, e.shape[1]), jnp.float32), esq,
            (((1,), (1,)), ((), ())),
            preferred_element_type=jnp.float32)
        counts = jax.lax.dot_general(
            jnp.ones((1, b), jnp.float32), oh, (((1,), (0,)), ((), ())),
            preferred_element_type=jnp.float32)
        cntr_ref[...] = jax.lax.dot_general(
            counts, oh, (((1,), (1,)), ((), ())),
            preferred_element_type=jnp.float32)
        loss_ref[...] = jnp.zeros((1, 1), jnp.float32)
        cnt_ref[...] = jnp.zeros((1, 1), jnp.int32)

    dot_idx = jnp.minimum(i, nblk - 1)
    red_idx = jnp.maximum(i - 1, 0)
    gate = i > 0

    @pl.when(i % 2 == 0)
    def _even():
        _dot_block(a_ref, c_ref, h0_ref, dot_idx, bc)
        _reduce_block(h1_ref, sqr_ref, cntr_ref, loss_ref, cnt_ref,
                      red_idx, gate, bc, b)

    @pl.when(i % 2 == 1)
    def _odd():
        _dot_block(a_ref, c_ref, h1_ref, dot_idx, bc)
        _reduce_block(h0_ref, sqr_ref, cntr_ref, loss_ref, cnt_ref,
                      red_idx, gate, bc, b)

    @pl.when(i == nblk)
    def _finalize():
        denom = jnp.maximum(cnt_ref[...].astype(jnp.float32), 1.0)
        loss_ref[...] = loss_ref[...] / denom


def kernel(embeddings, target):
    b, d = embeddings.shape
    bc = 1024
    nblk = b // bc
    t_row = target.astype(jnp.int32).reshape(1, b)

    body = functools.partial(_triplet_kernel, bc=bc, nblk=nblk)
    loss, cnt = pl.pallas_call(
        body,
        grid=(nblk + 1,),
        in_specs=[
            pl.BlockSpec(embeddings.shape, lambda i: (0, 0)),
            pl.BlockSpec(t_row.shape, lambda i: (0, 0)),
        ],
        out_specs=[
            pl.BlockSpec((1, 1), lambda i: (0, 0)),
            pl.BlockSpec((1, 1), lambda i: (0, 0)),
        ],
        out_shape=[
            jax.ShapeDtypeStruct((1, 1), jnp.float32),
            jax.ShapeDtypeStruct((1, 1), jnp.int32),
        ],
        scratch_shapes=[
            pltpu.VMEM((b, d + 1 + NUM_LABELS), jnp.bfloat16),
            pltpu.VMEM((b, d + 1 + NUM_LABELS), jnp.bfloat16),
            pltpu.VMEM((b, bc), jnp.float32),
            pltpu.VMEM((b, bc), jnp.float32),
            pltpu.VMEM((1, b), jnp.float32),
            pltpu.VMEM((1, b), jnp.float32),
        ],
    )(embeddings, t_row)

    return (loss.reshape(()), cnt.reshape(()))


# speedup vs baseline: 1.2981x; 1.2981x over previous
import functools

import jax
import jax.numpy as jnp
from jax.experimental import pallas as pl
from jax.experimental.pallas import tpu as pltpu

MARGIN = 1.0
BIG = 131072.0
NUM_LABELS = 64


def _triplet_kernel(emb_ref, trow_ref, loss_ref, cnt_ref,
                    a_ref, c_ref, sqr_ref, cntr_ref, *, bc, nblk):
    i = pl.program_id(0)
    b = emb_ref.shape[0]

    @pl.when(i == 0)
    def _build():
        e = emb_ref[...]
        esq = e * e
        sq = jnp.sum(esq, axis=1, keepdims=True)
        tf = trow_ref[...].astype(jnp.float32)
        t = jnp.transpose(tf)
        lbl = jax.lax.broadcasted_iota(
            jnp.int32, (1, NUM_LABELS), 1).astype(jnp.float32)
        oh = (t == lbl).astype(jnp.float32)
        ones_col = jnp.ones((b, 1), jnp.float32)
        a_ref[...] = jnp.concatenate(
            [-2.0 * e, ones_col, BIG * oh], axis=1).astype(jnp.bfloat16)
        c_ref[...] = jnp.concatenate(
            [e, sq, oh], axis=1).astype(jnp.bfloat16)
        sqr_ref[...] = jax.lax.dot_general(
            jnp.ones((1, e.shape[1]), jnp.float32), esq,
            (((1,), (1,)), ((), ())),
            preferred_element_type=jnp.float32)
        counts = jax.lax.dot_general(
            jnp.ones((1, b), jnp.float32), oh, (((1,), (0,)), ((), ())),
            preferred_element_type=jnp.float32)
        cntr_ref[...] = jax.lax.dot_general(
            counts, oh, (((1,), (1,)), ((), ())),
            preferred_element_type=jnp.float32)
        loss_ref[...] = jnp.zeros((1, 1), jnp.float32)
        cnt_ref[...] = jnp.zeros((1, 1), jnp.int32)

    a_blk = a_ref[pl.ds(i * bc, bc), :]
    ht = jax.lax.dot_general(
        c_ref[...], a_blk, (((1,), (1,)), ((), ())),
        preferred_element_type=jnp.float32)
    mx = jnp.max(ht, axis=0, keepdims=True)
    mn = jnp.min(ht, axis=0, keepdims=True)

    sq_blk = sqr_ref[:, pl.ds(i * bc, bc)]
    cnt_blk = cntr_ref[:, pl.ds(i * bc, bc)]
    pos_max = mx - BIG + sq_blk
    neg_min = mn + sq_blk
    valid = (cnt_blk >= 2.0) & (cnt_blk < float(b))
    vf = valid.astype(jnp.float32)
    losses = jnp.maximum(pos_max - neg_min + MARGIN, 0.0) * vf

    loss_ref[...] += jnp.sum(losses).reshape(1, 1)
    cnt_ref[...] += jnp.sum(valid.astype(jnp.int32)).reshape(1, 1)

    @pl.when(i == nblk - 1)
    def _finalize():
        denom = jnp.maximum(cnt_ref[...].astype(jnp.float32), 1.0)
        loss_ref[...] = loss_ref[...] / denom


def kernel(embeddings, target):
    b, d = embeddings.shape
    bc = 2048
    nblk = b // bc
    t_row = target.astype(jnp.int32).reshape(1, b)

    body = functools.partial(_triplet_kernel, bc=bc, nblk=nblk)
    loss, cnt = pl.pallas_call(
        body,
        grid=(nblk,),
        in_specs=[
            pl.BlockSpec(embeddings.shape, lambda i: (0, 0)),
            pl.BlockSpec(t_row.shape, lambda i: (0, 0)),
        ],
        out_specs=[
            pl.BlockSpec((1, 1), lambda i: (0, 0)),
            pl.BlockSpec((1, 1), lambda i: (0, 0)),
        ],
        out_shape=[
            jax.ShapeDtypeStruct((1, 1), jnp.float32),
            jax.ShapeDtypeStruct((1, 1), jnp.int32),
        ],
        scratch_shapes=[
            pltpu.VMEM((b, d + 1 + NUM_LABELS), jnp.bfloat16),
            pltpu.VMEM((b, d + 1 + NUM_LABELS), jnp.bfloat16),
            pltpu.VMEM((1, b), jnp.float32),
            pltpu.VMEM((1, b), jnp.float32),
        ],
    )(embeddings, t_row)

    return (loss.reshape(()), cnt.reshape(()))


# speedup vs baseline: 1.3215x; 1.0180x over previous
import functools

import jax
import jax.numpy as jnp
from jax.experimental import pallas as pl
from jax.experimental.pallas import tpu as pltpu

MARGIN = 1.0
BIG = 131072.0
NUM_LABELS = 64


def _triplet_kernel(emb_ref, trow_ref, loss_ref, cnt_ref,
                    a_ref, c_ref, sqr_ref, cntr_ref, *, bc, nblk):
    i = pl.program_id(0)
    b = emb_ref.shape[0]

    @pl.when(i == 0)
    def _build():
        e = emb_ref[...]
        esq = e * e
        sq = jax.lax.dot_general(
            esq, jnp.ones((e.shape[1], 1), jnp.float32),
            (((1,), (0,)), ((), ())),
            preferred_element_type=jnp.float32)
        tf = trow_ref[...].astype(jnp.float32)
        t = jnp.transpose(tf)
        lbl = jax.lax.broadcasted_iota(
            jnp.int32, (1, NUM_LABELS), 1).astype(jnp.float32)
        oh = (t == lbl).astype(jnp.float32)
        ones_col = jnp.ones((b, 1), jnp.float32)
        a_ref[...] = jnp.concatenate(
            [-2.0 * e, ones_col, BIG * oh], axis=1).astype(jnp.bfloat16)
        c_ref[...] = jnp.concatenate(
            [e, sq, oh], axis=1).astype(jnp.bfloat16)
        sqr_ref[...] = jax.lax.dot_general(
            jnp.ones((1, e.shape[1]), jnp.float32), esq,
            (((1,), (1,)), ((), ())),
            preferred_element_type=jnp.float32)
        counts = jax.lax.dot_general(
            jnp.ones((1, b), jnp.float32), oh, (((1,), (0,)), ((), ())),
            preferred_element_type=jnp.float32)
        cntr_ref[...] = jax.lax.dot_general(
            counts, oh, (((1,), (1,)), ((), ())),
            preferred_element_type=jnp.float32)
        loss_ref[...] = jnp.zeros((1, 1), jnp.float32)
        cnt_ref[...] = jnp.zeros((1, 1), jnp.int32)

    a_blk = a_ref[pl.ds(i * bc, bc), :]
    ht = jax.lax.dot_general(
        c_ref[...], a_blk, (((1,), (1,)), ((), ())),
        preferred_element_type=jnp.float32)
    mx = jnp.max(ht, axis=0, keepdims=True)
    mn = jnp.min(ht, axis=0, keepdims=True)

    sq_blk = sqr_ref[:, pl.ds(i * bc, bc)]
    cnt_blk = cntr_ref[:, pl.ds(i * bc, bc)]
    pos_max = mx - BIG + sq_blk
    neg_min = mn + sq_blk
    valid = (cnt_blk >= 2.0) & (cnt_blk < float(b))
    vf = valid.astype(jnp.float32)
    losses = jnp.maximum(pos_max - neg_min + MARGIN, 0.0) * vf

    loss_ref[...] += jnp.sum(losses).reshape(1, 1)
    cnt_ref[...] += jnp.sum(valid.astype(jnp.int32)).reshape(1, 1)

    @pl.when(i == nblk - 1)
    def _finalize():
        denom = jnp.maximum(cnt_ref[...].astype(jnp.float32), 1.0)
        loss_ref[...] = loss_ref[...] / denom


def kernel(embeddings, target):
    b, d = embeddings.shape
    bc = 2048
    nblk = b // bc
    t_row = target.astype(jnp.int32).reshape(1, b)

    body = functools.partial(_triplet_kernel, bc=bc, nblk=nblk)
    loss, cnt = pl.pallas_call(
        body,
        grid=(nblk,),
        in_specs=[
            pl.BlockSpec(embeddings.shape, lambda i: (0, 0)),
            pl.BlockSpec(t_row.shape, lambda i: (0, 0)),
        ],
        out_specs=[
            pl.BlockSpec((1, 1), lambda i: (0, 0)),
            pl.BlockSpec((1, 1), lambda i: (0, 0)),
        ],
        out_shape=[
            jax.ShapeDtypeStruct((1, 1), jnp.float32),
            jax.ShapeDtypeStruct((1, 1), jnp.int32),
        ],
        scratch_shapes=[
            pltpu.VMEM((b, d + 1 + NUM_LABELS), jnp.bfloat16),
            pltpu.VMEM((b, d + 1 + NUM_LABELS), jnp.bfloat16),
            pltpu.VMEM((1, b), jnp.float32),
            pltpu.VMEM((1, b), jnp.float32),
        ],
    )(embeddings, t_row)

    return (loss.reshape(()), cnt.reshape(()))
